# Initial kernel scaffold; baseline (speedup 1.0000x reference)
import functools

import jax
import jax.numpy as jnp
from jax import lax
from jax.experimental import pallas as pl
from jax.experimental.pallas import tpu as pltpu
from jax.experimental.pallas import tpu_sc as plsc

N_NODES = 10000
N_EDGES = 320000
D = 128

NC = 2
NS = 16
NW = NC * NS
NPW = 313
N_PAD = NW * NPW

CHUNK = 4000
NCHUNK = N_EDGES // CHUNK
GROUP = 16

_NEG_INF = jnp.float32(-jnp.inf)


def _sc_agg_kernel(h_hbm, src_hbm, dst_hbm, out_hbm,
                   dstv, srcv, hit_src, hit_dst, rows_v, agg, sem):
    wid = lax.axis_index("s") * NC + lax.axis_index("c")
    lo = wid * NPW
    hi = lo + NPW

    iota = lax.iota(jnp.int32, 16)

    neg = jnp.full((16,), _NEG_INF, jnp.float32)

    def init_body(i, _):
        agg[pl.ds(i * 16, 16)] = neg
        return 0

    lax.fori_loop(0, (NPW * D) // 16, init_body, 0)

    def chunk_body(ci, _):
        base = ci * CHUNK
        pltpu.sync_copy(dst_hbm.at[pl.ds(base, CHUNK)], dstv)
        pltpu.sync_copy(src_hbm.at[pl.ds(base, CHUNK)], srcv)

        def scan_body(i, cnt_v):
            d = dstv[pl.ds(i * 16, 16)]
            s = srcv[pl.ds(i * 16, 16)]
            m = (d >= lo) & (d < hi)
            mi = m.astype(jnp.int32)
            pos = cnt_v + plsc.cumsum(mi) - mi
            plsc.store_scatter(hit_src, [pos], s, mask=m)
            plsc.store_scatter(hit_dst, [pos], d - lo, mask=m)
            return cnt_v + plsc.all_reduce_population_count(m)

        cnt_v = lax.fori_loop(0, CHUNK // 16, scan_body,
                              jnp.zeros((16,), jnp.int32))
        cnt = jnp.max(cnt_v)

        ngroups = (cnt + (GROUP - 1)) // GROUP

        def group_body(g, _):
            gbase = g * GROUP
            idx = hit_src[pl.ds(gbase, GROUP)]
            idx = jnp.minimum(jnp.maximum(idx, 0), N_NODES - 1)
            pltpu.async_copy(h_hbm.at[idx], rows_v, sem).wait()
            dl = hit_dst[pl.ds(gbase, GROUP)]
            jmax = jnp.minimum(cnt - gbase, GROUP)

            def merge_body(j, _):
                d_s = jnp.sum(jnp.where(iota == j, dl, 0))
                lane_v = jnp.full((16,), j, jnp.int32)
                off = d_s * D
                for k in range(D // 16):
                    r = plsc.load_gather(rows_v, [lane_v, iota + (k * 16)])
                    a = agg[pl.ds(off + k * 16, 16)]
                    agg[pl.ds(off + k * 16, 16)] = jnp.maximum(a, r)
                return 0

            lax.fori_loop(0, jmax, merge_body, 0)
            return 0

        lax.fori_loop(0, ngroups, group_body, 0)
        return 0

    lax.fori_loop(0, NCHUNK, chunk_body, 0)

    pltpu.sync_copy(agg, out_hbm.at[pl.ds(lo * D, NPW * D)])


@jax.jit
def _sc_agg(h, src, dst):
    mesh = plsc.VectorSubcoreMesh(core_axis_name="c", subcore_axis_name="s")
    f = functools.partial(
        pl.kernel,
        mesh=mesh,
        out_type=jax.ShapeDtypeStruct((N_PAD * D,), jnp.float32),
        scratch_types=[
            pltpu.VMEM((CHUNK,), jnp.int32),
            pltpu.VMEM((CHUNK,), jnp.int32),
            pltpu.VMEM((CHUNK + 16,), jnp.int32),
            pltpu.VMEM((CHUNK + 16,), jnp.int32),
            pltpu.VMEM((GROUP, D), jnp.float32),
            pltpu.VMEM((NPW * D,), jnp.float32),
            pltpu.SemaphoreType.DMA,
        ],
    )(_sc_agg_kernel)
    return f(h, src, dst)


ROWS_BLK = 400


def _tc_tail_kernel(h_ref, agg_ref, wt_ref, b_ref, g_ref, be_ref, o_ref):
    ag = agg_ref[...]
    ag = jnp.where(ag == _NEG_INF, jnp.float32(0.0), ag)
    rst = h_ref[...] + ag
    x = jnp.dot(rst, wt_ref[...], preferred_element_type=jnp.float32)
    x = x + b_ref[...]
    x = jnp.maximum(x, jnp.float32(0.0))
    mean = jnp.mean(x, axis=1, keepdims=True)
    xc = x - mean
    var = jnp.mean(xc * xc, axis=1, keepdims=True)
    inv = lax.rsqrt(var + jnp.float32(1e-5))
    o_ref[...] = xc * inv * g_ref[...] + be_ref[...]


@jax.jit
def _tc_tail(h, agg, wt, b2, g2, be2):
    grid = N_NODES // ROWS_BLK
    return pl.pallas_call(
        _tc_tail_kernel,
        grid=(grid,),
        in_specs=[
            pl.BlockSpec((ROWS_BLK, D), lambda i: (i, 0)),
            pl.BlockSpec((ROWS_BLK, D), lambda i: (i, 0)),
            pl.BlockSpec((D, D), lambda i: (0, 0)),
            pl.BlockSpec((1, D), lambda i: (0, 0)),
            pl.BlockSpec((1, D), lambda i: (0, 0)),
            pl.BlockSpec((1, D), lambda i: (0, 0)),
        ],
        out_specs=pl.BlockSpec((ROWS_BLK, D), lambda i: (i, 0)),
        out_shape=jax.ShapeDtypeStruct((N_NODES, D), jnp.float32),
    )(h, agg, wt, b2, g2, be2)


def kernel(h, edge_index, W, b, ln_gamma, ln_beta):
    src = edge_index[0]
    dst = edge_index[1]
    agg = _sc_agg(h, src, dst).reshape(N_PAD, D)[:N_NODES]
    return _tc_tail(h, agg, W.T, b.reshape(1, D),
                    ln_gamma.reshape(1, D), ln_beta.reshape(1, D))


# speedup vs baseline: 1.5985x; 1.5985x over previous
import functools

import jax
import jax.numpy as jnp
from jax import lax
from jax.experimental import pallas as pl
from jax.experimental.pallas import tpu as pltpu
from jax.experimental.pallas import tpu_sc as plsc

N_NODES = 10000
N_EDGES = 320000
D = 128

NC = 2
NS = 16
NW = NC * NS
NPW = 313
N_PAD = NW * NPW

CHUNK = 4000
NCHUNK = N_EDGES // CHUNK
GROUP = 16
TRASH = CHUNK + 16

_NEG_INF = float("-inf")

_GDN = lax.GatherDimensionNumbers(
    offset_dims=(), collapsed_slice_dims=(0,), start_index_map=(0,))


def _lane_take(x, idx):
    return lax.gather(x, idx[:, None], dimension_numbers=_GDN,
                      slice_sizes=(1,),
                      mode=lax.GatherScatterMode.PROMISE_IN_BOUNDS)


def _sc_agg_kernel(h_hbm, src_hbm, dst_hbm, out_hbm,
                   dstv, srcv, hit_src, hit_dst, rows_v, agg, sem):
    wid = lax.axis_index("s") * NC + lax.axis_index("c")
    lo = wid * NPW
    hi = lo + NPW

    iota = lax.iota(jnp.int32, 16)

    neg = jnp.full((16,), _NEG_INF, jnp.float32)

    def init_body(i, _):
        agg[pl.ds(i * 16, 16)] = neg
        return 0

    lax.fori_loop(0, (NPW * D) // 16, init_body, 0)

    def chunk_body(ci, _):
        base = ci * CHUNK
        pltpu.sync_copy(dst_hbm.at[pl.ds(base, CHUNK)], dstv)
        pltpu.sync_copy(src_hbm.at[pl.ds(base, CHUNK)], srcv)

        def scan_body(i, cnt_v):
            d = dstv[pl.ds(i * 16, 16)]
            s = srcv[pl.ds(i * 16, 16)]
            m = (d >= lo) & (d < hi)
            mi = jnp.where(m, jnp.int32(1), jnp.int32(0))
            pos = cnt_v + plsc.cumsum(mi) - mi
            plsc.store_scatter(hit_src, [pos], s, mask=m)
            plsc.store_scatter(hit_dst, [pos], d - lo, mask=m)
            return cnt_v + plsc.all_reduce_population_count(m)

        cnt_v = lax.fori_loop(0, CHUNK // 16, scan_body,
                              jnp.zeros((16,), jnp.int32))
        cnt = cnt_v[0]

        ngroups = (cnt + (GROUP - 1)) // GROUP

        def group_body(g, _):
            gbase = g * GROUP
            idx = hit_src[pl.ds(gbase, GROUP)]
            idx = jnp.minimum(jnp.maximum(idx, 0), N_NODES - 1)
            pltpu.async_copy(h_hbm.at[idx], rows_v, sem).wait()
            dl = hit_dst[pl.ds(gbase, GROUP)]
            jmax = cnt - gbase

            for j in range(GROUP):
                @pl.when(j < jmax)
                def _merge(j=j):
                    d_s = dl[j]
                    off = d_s * D
                    lane_v = jnp.full((16,), j, jnp.int32)
                    for k in range(D // 16):
                        r = plsc.load_gather(rows_v, [lane_v, iota + (k * 16)])
                        a = agg[pl.ds(off + k * 16, 16)]
                        agg[pl.ds(off + k * 16, 16)] = jnp.maximum(a, r)

            return 0

        lax.fori_loop(0, ngroups, group_body, 0)
        return 0

    lax.fori_loop(0, NCHUNK, chunk_body, 0)

    pltpu.sync_copy(agg, out_hbm.at[pl.ds(lo * D, NPW * D)])


@jax.jit
def _sc_agg(h, src, dst):
    mesh = plsc.VectorSubcoreMesh(core_axis_name="c", subcore_axis_name="s")
    f = functools.partial(
        pl.kernel,
        mesh=mesh,
        compiler_params=pltpu.CompilerParams(needs_layout_passes=False),
        out_type=jax.ShapeDtypeStruct((N_PAD * D,), jnp.float32),
        scratch_types=[
            pltpu.VMEM((CHUNK,), jnp.int32),
            pltpu.VMEM((CHUNK,), jnp.int32),
            pltpu.VMEM((CHUNK + 24,), jnp.int32),
            pltpu.VMEM((CHUNK + 24,), jnp.int32),
            pltpu.VMEM((GROUP, D), jnp.float32),
            pltpu.VMEM((NPW * D,), jnp.float32),
            pltpu.SemaphoreType.DMA,
        ],
    )(_sc_agg_kernel)
    return f(h, src, dst)


ROWS_BLK = 400


def _tc_tail_kernel(h_ref, agg_ref, wt_ref, b_ref, g_ref, be_ref, o_ref):
    ag = agg_ref[...]
    ag = jnp.where(ag == _NEG_INF, jnp.float32(0.0), ag)
    rst = h_ref[...] + ag
    x = jnp.dot(rst, wt_ref[...], preferred_element_type=jnp.float32)
    x = x + b_ref[...]
    x = jnp.maximum(x, jnp.float32(0.0))
    mean = jnp.mean(x, axis=1, keepdims=True)
    xc = x - mean
    var = jnp.mean(xc * xc, axis=1, keepdims=True)
    inv = lax.rsqrt(var + jnp.float32(1e-5))
    o_ref[...] = xc * inv * g_ref[...] + be_ref[...]


@jax.jit
def _tc_tail(h, agg, wt, b2, g2, be2):
    grid = N_NODES // ROWS_BLK
    return pl.pallas_call(
        _tc_tail_kernel,
        grid=(grid,),
        in_specs=[
            pl.BlockSpec((ROWS_BLK, D), lambda i: (i, 0)),
            pl.BlockSpec((ROWS_BLK, D), lambda i: (i, 0)),
            pl.BlockSpec((D, D), lambda i: (0, 0)),
            pl.BlockSpec((1, D), lambda i: (0, 0)),
            pl.BlockSpec((1, D), lambda i: (0, 0)),
            pl.BlockSpec((1, D), lambda i: (0, 0)),
        ],
        out_specs=pl.BlockSpec((ROWS_BLK, D), lambda i: (i, 0)),
        out_shape=jax.ShapeDtypeStruct((N_NODES, D), jnp.float32),
    )(h, agg, wt, b2, g2, be2)


def kernel(h, edge_index, W, b, ln_gamma, ln_beta):
    src = edge_index[0]
    dst = edge_index[1]
    agg = _sc_agg(h, src, dst).reshape(N_PAD, D)[:N_NODES]
    return _tc_tail(h, agg, W.T, b.reshape(1, D),
                    ln_gamma.reshape(1, D), ln_beta.reshape(1, D))


# speedup vs baseline: 1.8894x; 1.1820x over previous
import functools

import jax
import jax.numpy as jnp
from jax import lax
from jax.experimental import pallas as pl
from jax.experimental.pallas import tpu as pltpu
from jax.experimental.pallas import tpu_sc as plsc

N_NODES = 10000
N_EDGES = 320000
D = 128

NC = 2
NS = 16
NPW = N_NODES // NS
E_HALF = N_EDGES // NC

CHUNK = 4000
NCHUNK = E_HALF // CHUNK
GROUP = 16

_NEG_INF = float("-inf")


def _sc_agg_kernel(h_hbm, src_hbm, dst_hbm, out_hbm,
                   dstv, srcv, hit_src, hit_dst, rows_v, agg, sem):
    c = lax.axis_index("c")
    s = lax.axis_index("s")
    lo = s * NPW
    hi = lo + NPW
    ebase = c * E_HALF

    iota = lax.iota(jnp.int32, 16)
    neg = jnp.full((16,), _NEG_INF, jnp.float32)

    def init_body(i, _):
        agg[pl.ds(i * 16, 16)] = neg
        return 0

    lax.fori_loop(0, (NPW * D) // 16, init_body, 0)

    def chunk_body(ci, _):
        base = ebase + ci * CHUNK
        pltpu.sync_copy(dst_hbm.at[pl.ds(base, CHUNK)], dstv)
        pltpu.sync_copy(src_hbm.at[pl.ds(base, CHUNK)], srcv)

        def scan_body(i, cnt_v):
            d = dstv[pl.ds(i * 16, 16)]
            sv = srcv[pl.ds(i * 16, 16)]
            m = (d >= lo) & (d < hi)
            mi = jnp.where(m, jnp.int32(1), jnp.int32(0))
            pos = cnt_v + plsc.cumsum(mi) - mi
            plsc.store_scatter(hit_src, [pos], sv, mask=m)
            plsc.store_scatter(hit_dst, [pos], d - lo, mask=m)
            return cnt_v + plsc.all_reduce_population_count(m)

        cnt_v = lax.fori_loop(0, CHUNK // 16, scan_body,
                              jnp.zeros((16,), jnp.int32))
        cnt = cnt_v[0]

        ngroups = (cnt + (GROUP - 1)) // GROUP

        def group_body(g, _):
            gbase = g * GROUP
            idx = hit_src[pl.ds(gbase, GROUP)]
            idx = jnp.minimum(jnp.maximum(idx, 0), N_NODES - 1)
            pltpu.async_copy(h_hbm.at[idx], rows_v, sem).wait()
            dl = hit_dst[pl.ds(gbase, GROUP)]
            jmax = cnt - gbase

            for j in range(GROUP):
                @pl.when(j < jmax)
                def _merge(j=j):
                    d_s = dl[j]
                    off = d_s * D
                    for k in range(D // 16):
                        r = rows_v[j, pl.ds(k * 16, 16)]
                        a = agg[pl.ds(off + k * 16, 16)]
                        agg[pl.ds(off + k * 16, 16)] = jnp.maximum(a, r)

            return 0

        lax.fori_loop(0, ngroups, group_body, 0)
        return 0

    lax.fori_loop(0, NCHUNK, chunk_body, 0)

    pltpu.sync_copy(agg, out_hbm.at[pl.ds((c * N_NODES + lo) * D, NPW * D)])


@jax.jit
def _sc_agg(h, src, dst):
    mesh = plsc.VectorSubcoreMesh(core_axis_name="c", subcore_axis_name="s")
    f = functools.partial(
        pl.kernel,
        mesh=mesh,
        compiler_params=pltpu.CompilerParams(needs_layout_passes=False),
        out_type=jax.ShapeDtypeStruct((NC * N_NODES * D,), jnp.float32),
        scratch_types=[
            pltpu.VMEM((CHUNK,), jnp.int32),
            pltpu.VMEM((CHUNK,), jnp.int32),
            pltpu.VMEM((CHUNK + 16,), jnp.int32),
            pltpu.VMEM((CHUNK + 16,), jnp.int32),
            pltpu.VMEM((GROUP, D), jnp.float32),
            pltpu.VMEM((NPW * D,), jnp.float32),
            pltpu.SemaphoreType.DMA,
        ],
    )(_sc_agg_kernel)
    return f(h, src, dst)


ROWS_BLK = 400


def _tc_tail_kernel(h_ref, a0_ref, a1_ref, wt_ref, b_ref, g_ref, be_ref,
                    o_ref):
    ag = jnp.maximum(a0_ref[...], a1_ref[...])
    ag = jnp.where(ag == _NEG_INF, jnp.float32(0.0), ag)
    rst = h_ref[...] + ag
    x = jnp.dot(rst, wt_ref[...], preferred_element_type=jnp.float32)
    x = x + b_ref[...]
    x = jnp.maximum(x, jnp.float32(0.0))
    mean = jnp.mean(x, axis=1, keepdims=True)
    xc = x - mean
    var = jnp.mean(xc * xc, axis=1, keepdims=True)
    inv = lax.rsqrt(var + jnp.float32(1e-5))
    o_ref[...] = xc * inv * g_ref[...] + be_ref[...]


@jax.jit
def _tc_tail(h, a0, a1, wt, b2, g2, be2):
    grid = N_NODES // ROWS_BLK
    blk = pl.BlockSpec((ROWS_BLK, D), lambda i: (i, 0))
    full = pl.BlockSpec((D, D), lambda i: (0, 0))
    row = pl.BlockSpec((1, D), lambda i: (0, 0))
    return pl.pallas_call(
        _tc_tail_kernel,
        grid=(grid,),
        in_specs=[blk, blk, blk, full, row, row, row],
        out_specs=blk,
        out_shape=jax.ShapeDtypeStruct((N_NODES, D), jnp.float32),
    )(h, a0, a1, wt, b2, g2, be2)


def kernel(h, edge_index, W, b, ln_gamma, ln_beta):
    src = edge_index[0]
    dst = edge_index[1]
    aggs = _sc_agg(h, src, dst).reshape(NC, N_NODES, D)
    return _tc_tail(h, aggs[0], aggs[1], W.T, b.reshape(1, D),
                    ln_gamma.reshape(1, D), ln_beta.reshape(1, D))
